# baseline (device time: 35192 ns/iter reference)
import jax
import jax.numpy as jnp
from jax import lax
from jax.experimental import pallas as pl
from jax.experimental.pallas import tpu as pltpu

N_DEV = 4
N_CHUNK = 2
N_SLOT = (N_DEV - 1) * N_CHUNK


def kernel(x, w_mat, scale_x, scale_w):
    m_per, k = x.shape
    _, n = w_mat.shape
    n_per = n // N_DEV
    m = m_per * N_DEV
    m_chunk = m_per // N_CHUNK

    def body(x_ref, w_ref, sx_ref, sw_ref, out_ref,
             sq_ref, ss_ref, rq_ref, rs_ref,
             dsend_sems, drecv_sems, ssend_sems, srecv_sems):
        my = lax.axis_index("i")

        barrier = pltpu.get_barrier_semaphore()
        for d in range(1, N_DEV):
            peer = lax.rem(my + d, N_DEV)
            pl.semaphore_signal(barrier, inc=1, device_id=(peer,),
                                device_id_type=pl.DeviceIdType.MESH)
        barrier_waited = [False]

        def wait_barrier_once():
            if not barrier_waited[0]:
                pl.semaphore_wait(barrier, N_DEV - 1)
                barrier_waited[0] = True

        scale = sx_ref[0] * sw_ref[0]

        def chunk(col_pos, h):
            acc = lax.dot_general(
                x_ref[pl.ds(h * m_chunk, m_chunk), :].astype(jnp.bfloat16),
                w_ref[:, pl.ds(col_pos * n_per, n_per)].astype(jnp.bfloat16),
                (((1,), (0,)), ((), ())),
                preferred_element_type=jnp.float32,
            )
            y = acc * scale
            return y * jax.nn.sigmoid(y)

        rdmas = []
        for d in (2, 1, 3):
            tgt = lax.rem(my + d, N_DEV)
            for h in range(N_CHUNK):
                slot = (d - 1) * N_CHUNK + h
                y = chunk(tgt, h)
                cmax = jnp.maximum(
                    jnp.max(jnp.abs(y), axis=0, keepdims=True), 1e-20)
                sq_ref[slot, :, :] = jnp.clip(
                    jnp.round(y * (127.0 / cmax)), -127.0, 127.0
                ).astype(jnp.int8)
                ss_ref[slot, :, :] = jnp.broadcast_to(
                    cmax * (1.0 / 127.0), (8, n_per))
                qd = pltpu.make_async_remote_copy(
                    src_ref=sq_ref.at[slot],
                    dst_ref=rq_ref.at[slot],
                    send_sem=dsend_sems.at[slot],
                    recv_sem=drecv_sems.at[slot],
                    device_id=(tgt,),
                    device_id_type=pl.DeviceIdType.MESH,
                )
                qs = pltpu.make_async_remote_copy(
                    src_ref=ss_ref.at[slot],
                    dst_ref=rs_ref.at[slot],
                    send_sem=ssend_sems.at[slot],
                    recv_sem=srecv_sems.at[slot],
                    device_id=(tgt,),
                    device_id_type=pl.DeviceIdType.MESH,
                )
                wait_barrier_once()
                qd.start()
                qs.start()
                rdmas.append((d, h, slot, qd, qs))

        def drain(entries):
            for d, h, slot, qd, qs in entries:
                src_pos = lax.rem(my - d + N_DEV, N_DEV)
                qd.wait_recv()
                qs.wait_recv()
                out_ref[pl.ds(src_pos * m_per + h * m_chunk, m_chunk), :] = (
                    rq_ref[slot, :, :].astype(jnp.float32)
                    * rs_ref[slot, 0:1, :]
                ).astype(jnp.bfloat16)

        out_ref[pl.ds(my * m_per, m_chunk), :] = (
            chunk(my, 0).astype(jnp.bfloat16))
        drain(rdmas[:N_CHUNK])
        out_ref[pl.ds(my * m_per + m_chunk, m_chunk), :] = (
            chunk(my, 1).astype(jnp.bfloat16))
        drain(rdmas[N_CHUNK:])
        for _, _, _, qd, qs in rdmas:
            qd.wait_send()
            qs.wait_send()

    return pl.pallas_call(
        body,
        out_shape=jax.ShapeDtypeStruct((m, n_per), jnp.bfloat16),
        in_specs=[
            pl.BlockSpec(memory_space=pltpu.VMEM),
            pl.BlockSpec(memory_space=pltpu.VMEM),
            pl.BlockSpec(memory_space=pltpu.SMEM),
            pl.BlockSpec(memory_space=pltpu.SMEM),
        ],
        out_specs=pl.BlockSpec(memory_space=pltpu.VMEM),
        scratch_shapes=[
            pltpu.VMEM((N_SLOT, m_chunk, n_per), jnp.int8),
            pltpu.VMEM((N_SLOT, 8, n_per), jnp.float32),
            pltpu.VMEM((N_SLOT, m_chunk, n_per), jnp.int8),
            pltpu.VMEM((N_SLOT, 8, n_per), jnp.float32),
            pltpu.SemaphoreType.DMA((N_SLOT,)),
            pltpu.SemaphoreType.DMA((N_SLOT,)),
            pltpu.SemaphoreType.DMA((N_SLOT,)),
            pltpu.SemaphoreType.DMA((N_SLOT,)),
        ],
        compiler_params=pltpu.CompilerParams(
            collective_id=0,
            vmem_limit_bytes=100 * 1024 * 1024,
        ),
    )(x, w_mat, scale_x, scale_w)
